# baseline (speedup 1.0000x reference)
import functools

import jax
import jax.numpy as jnp
from jax import lax
from jax.experimental import pallas as pl
from jax.experimental.pallas import tpu as pltpu
from jax.experimental.pallas import tpu_sc as plsc

_LVL_N = (16384, 4096, 1024, 256, 64)
_LVL_NP = (16384, 4096, 1024, 256, 128)
_CHUNK = (1024, 1024, 1024, 256, 64)
_NTOT = 21888
_NCAP = 3328
_NCV = _NCAP // 16
_OUTP = 112


def _decode_body(cls_ref, reg_ref, ctr_ref, pos_ref,
                 sc_ref, cl_ref, xy1_ref, xy2_ref):
    c = cls_ref[...]
    m = jnp.max(c, axis=-1)
    cls_iota = lax.broadcasted_iota(jnp.int32, c.shape, 2)
    am = jnp.min(jnp.where(c == m[..., None], cls_iota, c.shape[-1]), axis=-1)
    t = ctr_ref[...]
    score = jnp.sqrt(jax.nn.sigmoid(m) * jax.nn.sigmoid(t))
    r = reg_ref[...]
    p = pos_ref[...]
    ri = lax.broadcasted_iota(jnp.int32, r.shape, 2)
    pi = lax.broadcasted_iota(jnp.int32, p.shape, 2)
    ext_r = lambda k: jnp.exp(jnp.sum(jnp.where(ri == k, r, 0.0), axis=-1))
    ext_p = lambda k: jnp.sum(jnp.where(pi == k, p, 0.0), axis=-1)
    px, py = ext_p(0), ext_p(1)
    xmin = px - ext_r(0)
    ymin = py - ext_r(1)
    xmax = px + ext_r(2)
    ymax = py + ext_r(3)
    f = lambda v: v.astype(jnp.int32).astype(jnp.float32)
    x1 = jnp.maximum(f(xmin), 0.0)
    y1 = jnp.maximum(f(ymin), 0.0)
    x2 = jnp.minimum(f(xmax), 1023.0)
    y2 = jnp.minimum(f(ymax), 1023.0)
    sc_ref[...] = score
    cl_ref[...] = am.astype(jnp.float32)
    pk = lambda a, b: (a.astype(jnp.int32) & 0xFFFF) | (b.astype(jnp.int32) << 16)
    xy1_ref[...] = pk(x1, y1)
    xy2_ref[...] = pk(x2, y2)


def _decode_level(ch, rh, th, ph, n, chunk):
    b = ch.shape[0]
    grid = (n // chunk,)
    return pl.pallas_call(
        _decode_body,
        grid=grid,
        in_specs=[
            pl.BlockSpec((b, chunk, 80), lambda j: (0, j, 0)),
            pl.BlockSpec((b, chunk, 4), lambda j: (0, j, 0)),
            pl.BlockSpec((b, chunk), lambda j: (0, j)),
            pl.BlockSpec((b, chunk, 2), lambda j: (0, j, 0)),
        ],
        out_specs=[
            pl.BlockSpec((b, chunk), lambda j: (0, j)),
            pl.BlockSpec((b, chunk), lambda j: (0, j)),
            pl.BlockSpec((b, chunk), lambda j: (0, j)),
            pl.BlockSpec((b, chunk), lambda j: (0, j)),
        ],
        out_shape=[
            jax.ShapeDtypeStruct((b, n), jnp.float32),
            jax.ShapeDtypeStruct((b, n), jnp.float32),
            jax.ShapeDtypeStruct((b, n), jnp.int32),
            jax.ShapeDtypeStruct((b, n), jnp.int32),
        ],
    )(ch, rh, th, ph)


def _thresh_body(s0_ref, s1_ref, s2_ref, out_ref):
    lanes = lax.broadcasted_iota(jnp.int32, (8, 128), 1)
    refs = (s0_ref, s1_ref, s2_ref)

    def it_body(_, carry):
        lo, hi = carry
        mid = lax.shift_right_arithmetic(lo + hi + 1, 1)
        tf = lax.bitcast_convert_type(mid, jnp.float32)
        for l in range(3):
            t = lax.slice(tf, (0, l), (8, l + 1))
            s = refs[l][...]
            cnt = jnp.sum((s >= t).astype(jnp.int32), axis=1, keepdims=True)
            ge = cnt >= 1000
            lm = lanes == l
            lo = jnp.where(lm & ge, mid, lo)
            hi = jnp.where(lm & (~ge), mid - 1, hi)
        return lo, hi

    lo0 = jnp.zeros((8, 128), jnp.int32)
    hi0 = jnp.where(lanes < 3, 0x3F800000, 0)
    lo, _ = lax.fori_loop(0, 31, it_body, (lo0, hi0))
    acc = lo
    tfv = lax.bitcast_convert_type(lo, jnp.float32)
    for l in range(3):
        t = lax.slice(tfv, (0, l), (8, l + 1))
        s = refs[l][...]
        cgt = jnp.sum((s > t).astype(jnp.int32), axis=1, keepdims=True)
        acc = acc + jnp.where(lanes == (3 + l), cgt, 0)
    out_ref[...] = acc


def _make_nms_kernel():
    mesh = plsc.VectorSubcoreMesh(core_axis_name="c", subcore_axis_name="s")

    @functools.partial(
        pl.kernel,
        mesh=mesh,
        compiler_params=pltpu.CompilerParams(needs_layout_passes=False),
        out_type=[
            jax.ShapeDtypeStruct((8, _OUTP), jnp.float32),
            jax.ShapeDtypeStruct((8, _OUTP), jnp.float32),
            jax.ShapeDtypeStruct((8, _OUTP * 4), jnp.float32),
        ],
        scratch_types=[
            pltpu.VMEM((_NTOT,), jnp.float32),
            pltpu.VMEM((_NTOT,), jnp.int32),
            pltpu.VMEM((_NTOT,), jnp.int32),
            pltpu.VMEM((_NTOT,), jnp.float32),
            pltpu.VMEM((128,), jnp.int32),
            pltpu.VMEM((_NCAP,), jnp.float32),
            pltpu.VMEM((_NCAP,), jnp.int32),
            pltpu.VMEM((_NCAP,), jnp.int32),
            pltpu.VMEM((_NCAP,), jnp.float32),
            pltpu.VMEM((_NCAP,), jnp.float32),
            pltpu.VMEM((_NCAP,), jnp.float32),
            pltpu.VMEM((_NCAP,), jnp.float32),
            pltpu.VMEM((_NCAP,), jnp.float32),
            pltpu.VMEM((_NCAP,), jnp.float32),
            pltpu.VMEM((_OUTP,), jnp.float32),
            pltpu.VMEM((_OUTP,), jnp.float32),
            pltpu.VMEM((_OUTP * 4,), jnp.float32),
        ],
    )
    def nms_kernel(sc0, sc1, sc2, sc3, sc4,
                   xa0, xa1, xa2, xa3, xa4,
                   xb0, xb1, xb2, xb3, xb4,
                   cl0, cl1, cl2, cl3, cl4,
                   thr_hbm,
                   outs_hbm, outc_hbm, outb_hbm,
                   sc_v, xy1_v, xy2_v, cls_v, thr_v,
                   csc, cxy1, cxy2, ccls,
                   cx1, cy1, cx2, cy2, car,
                   os_v, oc_v, ob_v):
        wid = lax.axis_index("s") * 2 + lax.axis_index("c")

        @pl.when(wid < 8)
        def _():
            i = wid
            off = 0
            for l, (shb, xahb, xbhb, clhb) in enumerate(
                    zip((sc0, sc1, sc2, sc3, sc4),
                        (xa0, xa1, xa2, xa3, xa4),
                        (xb0, xb1, xb2, xb3, xb4),
                        (cl0, cl1, cl2, cl3, cl4))):
                nl = _LVL_NP[l]
                pltpu.sync_copy(shb.at[i], sc_v.at[pl.ds(off, nl)])
                pltpu.sync_copy(xahb.at[i], xy1_v.at[pl.ds(off, nl)])
                pltpu.sync_copy(xbhb.at[i], xy2_v.at[pl.ds(off, nl)])
                pltpu.sync_copy(clhb.at[i], cls_v.at[pl.ds(off, nl)])
                off += nl
            pltpu.sync_copy(thr_hbm.at[i], thr_v)

            lane = lax.iota(jnp.int32, 16)
            neg1 = jnp.full((16,), -1.0, jnp.float32)

            def init_csc(j4, carry):
                for u in range(4):
                    csc[pl.ds((j4 * 4 + u) * 16, 16)] = neg1
                return carry
            lax.fori_loop(0, _NCV // 4, init_csc, 0)

            def init_out(j, carry):
                os_v[pl.ds(j * 16, 16)] = neg1
                oc_v[pl.ds(j * 16, 16)] = neg1
                return carry
            lax.fori_loop(0, _OUTP // 16, init_out, 0)

            def init_ob(j, carry):
                ob_v[pl.ds(j * 16, 16)] = neg1
                return carry
            lax.fori_loop(0, _OUTP * 4 // 16, init_ob, 0)

            tv = thr_v[pl.ds(0, 16)]

            def lane_scalar(vec, l):
                return jnp.sum(jnp.where(lane == l, vec, 0))

            voff = 0
            woff = jnp.int32(0)
            for l, nl in enumerate(_LVL_NP):
                nv = nl // 16
                if nl >= 1000:
                    tbits = lane_scalar(tv, l)
                    ngt = lane_scalar(tv, 3 + l)
                    budget0 = jnp.int32(1000) - ngt
                    tf = lax.bitcast_convert_type(
                        jnp.zeros((16,), jnp.int32) + tbits, jnp.float32)

                    def scan_body(j2, carry, tf=tf):
                        w, budget = carry
                        for u in range(2):
                            j = j2 * 2 + u
                            s = sc_v[pl.ds(j * 16, 16)]
                            gt = s > tf
                            eq = s == tf
                            eqc = jnp.cumsum(eq.astype(jnp.int32))
                            sel_eq = eq & (eqc <= (jnp.zeros((16,), jnp.int32) + budget))
                            sel = gt | sel_eq
                            budget = budget - jnp.sum(sel_eq.astype(jnp.int32))
                            keep = sel & (s > 0.01)
                            ki = keep.astype(jnp.int32)
                            idx = (jnp.zeros((16,), jnp.int32) + w) + jnp.cumsum(ki) - 1
                            plsc.store_scatter(csc, [idx], s, mask=keep)
                            plsc.store_scatter(cxy1, [idx], xy1_v[pl.ds(j * 16, 16)], mask=keep)
                            plsc.store_scatter(cxy2, [idx], xy2_v[pl.ds(j * 16, 16)], mask=keep)
                            plsc.store_scatter(ccls, [idx], cls_v[pl.ds(j * 16, 16)], mask=keep)
                            w = w + jnp.sum(ki)
                        return w, budget

                    woff, _ = lax.fori_loop(voff // 2, (voff + nv) // 2,
                                            scan_body, (woff, budget0))
                else:
                    def scan_small(j2, w):
                        for u in range(2):
                            j = j2 * 2 + u
                            s = sc_v[pl.ds(j * 16, 16)]
                            keep = s > 0.01
                            ki = keep.astype(jnp.int32)
                            idx = (jnp.zeros((16,), jnp.int32) + w) + jnp.cumsum(ki) - 1
                            plsc.store_scatter(csc, [idx], s, mask=keep)
                            plsc.store_scatter(cxy1, [idx], xy1_v[pl.ds(j * 16, 16)], mask=keep)
                            plsc.store_scatter(cxy2, [idx], xy2_v[pl.ds(j * 16, 16)], mask=keep)
                            plsc.store_scatter(ccls, [idx], cls_v[pl.ds(j * 16, 16)], mask=keep)
                            w = w + jnp.sum(ki)
                        return w

                    woff = lax.fori_loop(voff // 2, (voff + nv) // 2,
                                         scan_small, woff)
                voff += nv

            def unpack_body(j4, carry):
                for u in range(4):
                    j = j4 * 4 + u
                    p1 = cxy1[pl.ds(j * 16, 16)]
                    p2 = cxy2[pl.ds(j * 16, 16)]
                    x1 = lax.shift_right_arithmetic(lax.shift_left(p1, 16), 16).astype(jnp.float32)
                    y1 = lax.shift_right_arithmetic(p1, 16).astype(jnp.float32)
                    x2 = lax.shift_right_arithmetic(lax.shift_left(p2, 16), 16).astype(jnp.float32)
                    y2 = lax.shift_right_arithmetic(p2, 16).astype(jnp.float32)
                    cx1[pl.ds(j * 16, 16)] = x1
                    cy1[pl.ds(j * 16, 16)] = y1
                    cx2[pl.ds(j * 16, 16)] = x2
                    cy2[pl.ds(j * 16, 16)] = y2
                    car[pl.ds(j * 16, 16)] = (x2 - x1) * (y2 - y1)
                return carry
            lax.fori_loop(0, _NCV // 4, unpack_body, 0)

            def amax_body(j4, carry):
                b, bj = carry
                for u in range(4):
                    j = j4 * 4 + u
                    m = jnp.max(csc[pl.ds(j * 16, 16)])
                    better = m > b
                    b = jnp.where(better, m, b)
                    bj = jnp.where(better, j, bj)
                return b, bj

            best0, bestj0 = lax.fori_loop(0, _NCV // 4, amax_body,
                                          (jnp.float32(-2.0), jnp.int32(0)))

            def cond_fn(st):
                kept, b, bj = st
                return (kept < 100) & (b >= 0.0)

            def body_fn(st):
                kept, b, bj = st
                bv = jnp.zeros((16,), jnp.float32) + b
                v = csc[pl.ds(bj * 16, 16)]
                eqm = v == bv
                lidx = jnp.zeros((16,), jnp.int32) + plsc.all_reduce_ffs(eqm)
                idxv = (jnp.zeros((16,), jnp.int32) + bj * 16) + lidx
                wx1 = plsc.load_gather(cx1, [idxv])
                wy1 = plsc.load_gather(cy1, [idxv])
                wx2 = plsc.load_gather(cx2, [idxv])
                wy2 = plsc.load_gather(cy2, [idxv])
                war = plsc.load_gather(car, [idxv])
                wcl = plsc.load_gather(ccls, [idxv])
                l0 = lane == 0
                kv = jnp.zeros((16,), jnp.int32) + kept
                plsc.store_scatter(os_v, [kv], bv, mask=l0)
                plsc.store_scatter(oc_v, [kv], wcl, mask=l0)
                bvals = jnp.where(lane == 0, wx1,
                        jnp.where(lane == 1, wy1,
                        jnp.where(lane == 2, wx2, wy2)))
                plsc.store_scatter(ob_v, [kv * 4 + lane], bvals, mask=lane < 4)
                plsc.store_scatter(csc, [idxv], neg1, mask=l0)

                def sup_body(j4, c2):
                    b2, bj2 = c2
                    for u in range(8):
                        j = j4 * 8 + u
                        s = csc[pl.ds(j * 16, 16)]
                        x1 = cx1[pl.ds(j * 16, 16)]
                        y1 = cy1[pl.ds(j * 16, 16)]
                        x2 = cx2[pl.ds(j * 16, 16)]
                        y2 = cy2[pl.ds(j * 16, 16)]
                        ar = car[pl.ds(j * 16, 16)]
                        xx1 = jnp.maximum(wx1, x1)
                        yy1 = jnp.maximum(wy1, y1)
                        xx2 = jnp.minimum(wx2, x2)
                        yy2 = jnp.minimum(wy2, y2)
                        w = jnp.maximum(xx2 - xx1, 0.0)
                        h = jnp.maximum(yy2 - yy1, 0.0)
                        inter = w * h
                        iou = inter / (war + ar - inter + 1e-12)
                        snew = jnp.where(iou > 0.6, neg1, s)
                        csc[pl.ds(j * 16, 16)] = snew
                        m = jnp.max(snew)
                        better = m > b2
                        b2 = jnp.where(better, m, b2)
                        bj2 = jnp.where(better, j, bj2)
                    return b2, bj2

                b3, bj3 = lax.fori_loop(0, _NCV // 8, sup_body,
                                        (jnp.float32(-2.0), jnp.int32(0)))
                return kept + 1, b3, bj3

            lax.while_loop(cond_fn, body_fn, (jnp.int32(0), best0, bestj0))

            pltpu.sync_copy(os_v, outs_hbm.at[i])
            pltpu.sync_copy(oc_v, outc_hbm.at[i])
            pltpu.sync_copy(ob_v, outb_hbm.at[i])

    return nms_kernel


_nms_call = _make_nms_kernel()


def kernel(cls_head_0, reg_head_0, center_head_0, positions_0,
           cls_head_1, reg_head_1, center_head_1, positions_1,
           cls_head_2, reg_head_2, center_head_2, positions_2,
           cls_head_3, reg_head_3, center_head_3, positions_3,
           cls_head_4, reg_head_4, center_head_4, positions_4):
    heads = [
        (cls_head_0, reg_head_0, center_head_0, positions_0),
        (cls_head_1, reg_head_1, center_head_1, positions_1),
        (cls_head_2, reg_head_2, center_head_2, positions_2),
        (cls_head_3, reg_head_3, center_head_3, positions_3),
        (cls_head_4, reg_head_4, center_head_4, positions_4),
    ]
    scs, cls_, xy1s, xy2s = [], [], [], []
    for l, (ch, rh, th, ph) in enumerate(heads):
        n = _LVL_N[l]
        b = ch.shape[0]
        sc, cl, xy1, xy2 = _decode_level(
            ch.reshape(b, n, ch.shape[-1]),
            rh.reshape(b, n, 4),
            th.reshape(b, n),
            ph.reshape(b, n, 2),
            n, _CHUNK[l])
        scs.append(sc)
        cls_.append(cl)
        xy1s.append(xy1)
        xy2s.append(xy2)
    thr = pl.pallas_call(
        _thresh_body,
        out_shape=jax.ShapeDtypeStruct((8, 128), jnp.int32),
    )(scs[0], scs[1], scs[2])
    pad_s = jnp.full((8, 64), -1.0, jnp.float32)
    pad_i = jnp.zeros((8, 64), jnp.int32)
    scs[4] = jnp.concatenate([scs[4], pad_s], axis=1)
    cls_[4] = jnp.concatenate([cls_[4], pad_s], axis=1)
    xy1s[4] = jnp.concatenate([xy1s[4], pad_i], axis=1)
    xy2s[4] = jnp.concatenate([xy2s[4], pad_i], axis=1)
    outs, outc, outb = _nms_call(*scs, *xy1s, *xy2s, *cls_, thr)
    outb = outb.reshape(8, _OUTP, 4)
    return outs[:, :100], outc[:, :100], outb[:, :100, :]


# speedup vs baseline: 2.5229x; 2.5229x over previous
import functools

import jax
import jax.numpy as jnp
from jax import lax
from jax.experimental import pallas as pl
from jax.experimental.pallas import tpu as pltpu
from jax.experimental.pallas import tpu_sc as plsc

_LVL_N = (16384, 4096, 1024, 256, 64)
_LVL_NP = (16384, 4096, 1024, 256, 128)
_CHUNK = (2048, 2048, 1024, 256, 64)
_NTOT = 21888
_NCAP = 3328
_NCV = _NCAP // 16
_QN = _NCAP // 4
_QV = _NCV // 4
_OUTP = 112


def _decode_body(cls_ref, reg_ref, ctr_ref, pos_ref,
                 sc_ref, cl_ref, xy1_ref, xy2_ref):
    c = cls_ref[...]
    m = jnp.max(c, axis=-1)
    cls_iota = lax.broadcasted_iota(jnp.int32, c.shape, 2)
    am = jnp.min(jnp.where(c == m[..., None], cls_iota, c.shape[-1]), axis=-1)
    t = ctr_ref[...]
    score = jnp.sqrt(jax.nn.sigmoid(m) * jax.nn.sigmoid(t))
    r = jnp.exp(reg_ref[...])
    p = pos_ref[...]
    xmin = p[:, 0, :] - r[:, 0, :]
    ymin = p[:, 1, :] - r[:, 1, :]
    xmax = p[:, 0, :] + r[:, 2, :]
    ymax = p[:, 1, :] + r[:, 3, :]
    f = lambda v: v.astype(jnp.int32).astype(jnp.float32)
    x1 = jnp.maximum(f(xmin), 0.0)
    y1 = jnp.maximum(f(ymin), 0.0)
    x2 = jnp.minimum(f(xmax), 1023.0)
    y2 = jnp.minimum(f(ymax), 1023.0)
    sc_ref[...] = score
    cl_ref[...] = am.astype(jnp.float32)
    pk = lambda a, b: (a.astype(jnp.int32) & 0xFFFF) | (b.astype(jnp.int32) << 16)
    xy1_ref[...] = pk(x1, y1)
    xy2_ref[...] = pk(x2, y2)


def _decode_level(ch, rh, th, ph, n, chunk):
    b = ch.shape[0]
    grid = (n // chunk,)
    return pl.pallas_call(
        _decode_body,
        grid=grid,
        in_specs=[
            pl.BlockSpec((b, chunk, 80), lambda j: (0, j, 0)),
            pl.BlockSpec((b, 4, chunk), lambda j: (0, 0, j)),
            pl.BlockSpec((b, chunk), lambda j: (0, j)),
            pl.BlockSpec((b, 2, chunk), lambda j: (0, 0, j)),
        ],
        out_specs=[
            pl.BlockSpec((b, chunk), lambda j: (0, j)),
            pl.BlockSpec((b, chunk), lambda j: (0, j)),
            pl.BlockSpec((b, chunk), lambda j: (0, j)),
            pl.BlockSpec((b, chunk), lambda j: (0, j)),
        ],
        out_shape=[
            jax.ShapeDtypeStruct((b, n), jnp.float32),
            jax.ShapeDtypeStruct((b, n), jnp.float32),
            jax.ShapeDtypeStruct((b, n), jnp.int32),
            jax.ShapeDtypeStruct((b, n), jnp.int32),
        ],
    )(ch, rh, th, ph)


def _thresh_body(s0_ref, s1_ref, s2_ref, out_ref):
    lanes = lax.broadcasted_iota(jnp.int32, (8, 128), 1)
    refs = (s0_ref, s1_ref, s2_ref)

    def it_body(_, carry):
        lo, hi = carry
        mid = lax.shift_right_arithmetic(lo + hi + 1, 1)
        tf = lax.bitcast_convert_type(mid, jnp.float32)
        for l in range(3):
            t = lax.slice(tf, (0, l), (8, l + 1))
            s = refs[l][...]
            cnt = jnp.sum((s >= t).astype(jnp.int32), axis=1, keepdims=True)
            ge = cnt >= 1000
            lm = lanes == l
            lo = jnp.where(lm & ge, mid, lo)
            hi = jnp.where(lm & (~ge), mid - 1, hi)
        return lo, hi

    lo0 = jnp.zeros((8, 128), jnp.int32)
    hi0 = jnp.where(lanes < 3, 0x3F800000, 0)
    lo, _ = lax.fori_loop(0, 31, it_body, (lo0, hi0))
    acc = lo
    tfv = lax.bitcast_convert_type(lo, jnp.float32)
    for l in range(3):
        t = lax.slice(tfv, (0, l), (8, l + 1))
        s = refs[l][...]
        cgt = jnp.sum((s > t).astype(jnp.int32), axis=1, keepdims=True)
        acc = acc + jnp.where(lanes == (3 + l), cgt, 0)
    out_ref[...] = acc


def _make_nms_kernel():
    mesh = plsc.VectorSubcoreMesh(core_axis_name="c", subcore_axis_name="s")

    @functools.partial(
        pl.kernel,
        mesh=mesh,
        compiler_params=pltpu.CompilerParams(needs_layout_passes=False),
        out_type=[
            jax.ShapeDtypeStruct((8, _OUTP), jnp.float32),
            jax.ShapeDtypeStruct((8, _OUTP), jnp.float32),
            jax.ShapeDtypeStruct((8, _OUTP * 4), jnp.float32),
        ],
        scratch_types=[
            pltpu.VMEM((_NTOT,), jnp.float32),
            pltpu.VMEM((_NTOT,), jnp.int32),
            pltpu.VMEM((_NTOT,), jnp.int32),
            pltpu.VMEM((_NTOT,), jnp.float32),
            pltpu.VMEM((128,), jnp.int32),
            pltpu.VMEM((_NCAP,), jnp.float32),
            pltpu.VMEM((_NCAP,), jnp.int32),
            pltpu.VMEM((_NCAP,), jnp.int32),
            pltpu.VMEM((_NCAP,), jnp.float32),
            pltpu.VMEM((_NCAP,), jnp.float32),
            pltpu.VMEM((_NCAP,), jnp.float32),
            pltpu.VMEM((_NCAP,), jnp.float32),
            pltpu.VMEM((_NCAP,), jnp.float32),
            pltpu.VMEM((_NCAP,), jnp.float32),
            pltpu.VMEM((_OUTP,), jnp.float32),
            pltpu.VMEM((_OUTP,), jnp.float32),
            pltpu.VMEM((_OUTP * 4,), jnp.float32),
            pltpu.VMEM((64,), jnp.float32),
            pltpu.VMEM((16,), jnp.float32),
            pltpu.VMEM_SHARED((4 * _NCAP,), jnp.float32),
            pltpu.VMEM_SHARED((4 * _NCAP,), jnp.int32),
            pltpu.VMEM_SHARED((4 * _NCAP,), jnp.int32),
            pltpu.VMEM_SHARED((512,), jnp.float32),
        ],
    )
    def nms_kernel(sc0, sc1, sc2, sc3, sc4,
                   xa0, xa1, xa2, xa3, xa4,
                   xb0, xb1, xb2, xb3, xb4,
                   cl0, cl1, cl2, cl3, cl4,
                   thr_hbm,
                   outs_hbm, outc_hbm, outb_hbm,
                   sc_v, xy1_v, xy2_v, cls_v, thr_v,
                   csc, cxy1, cxy2, ccls,
                   cx1, cy1, cx2, cy2, car,
                   os_v, oc_v, ob_v,
                   slotb, pubb, st_sc, st_x1, st_x2, slots):
        cc = lax.axis_index("c")
        ss = lax.axis_index("s")
        img = cc * 4 + ss // 4
        gl = ss // 4
        qq = ss % 4

        lane = lax.iota(jnp.int32, 16)
        neg1 = jnp.full((16,), -1.0, jnp.float32)

        def init_csc(j4, carry):
            for u in range(4):
                csc[pl.ds((j4 * 4 + u) * 16, 16)] = neg1
            return carry

        @pl.when(qq == 0)
        def _():
            i = img
            off = 0
            for l, (shb, xahb, xbhb, clhb) in enumerate(
                    zip((sc0, sc1, sc2, sc3, sc4),
                        (xa0, xa1, xa2, xa3, xa4),
                        (xb0, xb1, xb2, xb3, xb4),
                        (cl0, cl1, cl2, cl3, cl4))):
                nl = _LVL_NP[l]
                pltpu.sync_copy(shb.at[i], sc_v.at[pl.ds(off, nl)])
                pltpu.sync_copy(xahb.at[i], xy1_v.at[pl.ds(off, nl)])
                pltpu.sync_copy(xbhb.at[i], xy2_v.at[pl.ds(off, nl)])
                pltpu.sync_copy(clhb.at[i], cls_v.at[pl.ds(off, nl)])
                off += nl
            pltpu.sync_copy(thr_hbm.at[i], thr_v)

            lax.fori_loop(0, _NCV // 4, init_csc, 0)

            def init_out(j, carry):
                os_v[pl.ds(j * 16, 16)] = neg1
                oc_v[pl.ds(j * 16, 16)] = neg1
                return carry
            lax.fori_loop(0, _OUTP // 16, init_out, 0)

            def init_ob(j, carry):
                ob_v[pl.ds(j * 16, 16)] = neg1
                return carry
            lax.fori_loop(0, _OUTP * 4 // 16, init_ob, 0)

            tv = thr_v[pl.ds(0, 16)]

            def lane_scalar(vec, l):
                return jnp.sum(jnp.where(lane == l, vec, 0))

            voff = 0
            woff = jnp.int32(0)
            for l, nl in enumerate(_LVL_NP):
                nv = nl // 16
                if nl >= 1000:
                    tbits = lane_scalar(tv, l)
                    ngt = lane_scalar(tv, 3 + l)
                    budget0 = jnp.int32(1000) - ngt
                    tf = lax.bitcast_convert_type(
                        jnp.zeros((16,), jnp.int32) + tbits, jnp.float32)

                    def scan_body(j2, carry, tf=tf):
                        w, budget = carry
                        for u in range(2):
                            j = j2 * 2 + u
                            s = sc_v[pl.ds(j * 16, 16)]
                            gt = s > tf
                            eq = s == tf
                            eqc = jnp.cumsum(eq.astype(jnp.int32))
                            sel_eq = eq & (eqc <= (jnp.zeros((16,), jnp.int32) + budget))
                            sel = gt | sel_eq
                            budget = budget - jnp.sum(sel_eq.astype(jnp.int32))
                            keep = sel & (s > 0.01)
                            ki = keep.astype(jnp.int32)
                            idx = (jnp.zeros((16,), jnp.int32) + w) + jnp.cumsum(ki) - 1
                            plsc.store_scatter(csc, [idx], s, mask=keep)
                            plsc.store_scatter(cxy1, [idx], xy1_v[pl.ds(j * 16, 16)], mask=keep)
                            plsc.store_scatter(cxy2, [idx], xy2_v[pl.ds(j * 16, 16)], mask=keep)
                            plsc.store_scatter(ccls, [idx], cls_v[pl.ds(j * 16, 16)], mask=keep)
                            w = w + jnp.sum(ki)
                        return w, budget

                    woff, _ = lax.fori_loop(voff // 2, (voff + nv) // 2,
                                            scan_body, (woff, budget0))
                else:
                    def scan_small(j2, w):
                        for u in range(2):
                            j = j2 * 2 + u
                            s = sc_v[pl.ds(j * 16, 16)]
                            keep = s > 0.01
                            ki = keep.astype(jnp.int32)
                            idx = (jnp.zeros((16,), jnp.int32) + w) + jnp.cumsum(ki) - 1
                            plsc.store_scatter(csc, [idx], s, mask=keep)
                            plsc.store_scatter(cxy1, [idx], xy1_v[pl.ds(j * 16, 16)], mask=keep)
                            plsc.store_scatter(cxy2, [idx], xy2_v[pl.ds(j * 16, 16)], mask=keep)
                            plsc.store_scatter(ccls, [idx], cls_v[pl.ds(j * 16, 16)], mask=keep)
                            w = w + jnp.sum(ki)
                        return w

                    woff = lax.fori_loop(voff // 2, (voff + nv) // 2,
                                         scan_small, woff)
                voff += nv

            pltpu.sync_copy(csc, st_sc.at[pl.ds(gl * _NCAP, _NCAP)])
            pltpu.sync_copy(cxy1, st_x1.at[pl.ds(gl * _NCAP, _NCAP)])
            pltpu.sync_copy(cxy2, st_x2.at[pl.ds(gl * _NCAP, _NCAP)])

        plsc.subcore_barrier()

        lax.fori_loop(0, _NCV // 4, init_csc, 0)
        pltpu.sync_copy(st_x1.at[pl.ds(gl * _NCAP, _NCAP)], cxy1)
        pltpu.sync_copy(st_x2.at[pl.ds(gl * _NCAP, _NCAP)], cxy2)
        pltpu.sync_copy(st_sc.at[pl.ds(gl * _NCAP + qq * _QN, _QN)],
                        csc.at[pl.ds(qq * _QN, _QN)])

        def unpack_body(j4, carry):
            for u in range(4):
                j = j4 * 4 + u
                p1 = cxy1[pl.ds(j * 16, 16)]
                p2 = cxy2[pl.ds(j * 16, 16)]
                x1 = lax.shift_right_arithmetic(lax.shift_left(p1, 16), 16).astype(jnp.float32)
                y1 = lax.shift_right_arithmetic(p1, 16).astype(jnp.float32)
                x2 = lax.shift_right_arithmetic(lax.shift_left(p2, 16), 16).astype(jnp.float32)
                y2 = lax.shift_right_arithmetic(p2, 16).astype(jnp.float32)
                cx1[pl.ds(j * 16, 16)] = x1
                cy1[pl.ds(j * 16, 16)] = y1
                cx2[pl.ds(j * 16, 16)] = x2
                cy2[pl.ds(j * 16, 16)] = y2
                car[pl.ds(j * 16, 16)] = (x2 - x1) * (y2 - y1)
            return carry
        lax.fori_loop(0, _NCV // 4, unpack_body, 0)

        def publish(b, bj, buf):
            v = csc[pl.ds(bj * 16, 16)]
            eqm = v == (jnp.zeros((16,), jnp.float32) + b)
            lidx = jnp.zeros((16,), jnp.int32) + plsc.all_reduce_ffs(eqm)
            gidx = (jnp.zeros((16,), jnp.int32) + bj * 16) + lidx
            pub = jnp.where(lane == 0, jnp.zeros((16,), jnp.float32) + b,
                  jnp.where(lane == 1, plsc.bitcast(gidx, jnp.float32), 0.0))
            pubb[pl.ds(0, 16)] = pub
            pltpu.sync_copy(pubb, slots.at[pl.ds(buf * 256 + ss * 16, 16)])

        def amax_body(j4, carry):
            b, bj = carry
            for u in range(4):
                j = j4 * 4 + u
                m = jnp.max(csc[pl.ds(j * 16, 16)])
                better = m > b
                b = jnp.where(better, m, b)
                bj = jnp.where(better, j, bj)
            return b, bj

        best0, bestj0 = lax.fori_loop(qq * (_QV // 4), (qq + 1) * (_QV // 4),
                                      amax_body,
                                      (jnp.float32(-2.0), jnp.int32(0)))
        publish(best0, bestj0, 0)
        plsc.subcore_barrier()

        def keep_body(k, carry):
            p = k & 1
            pltpu.sync_copy(slots.at[pl.ds(p * 256 + gl * 64, 64)], slotb)
            best = jnp.float32(-2.0)
            bidx = jnp.int32(0)
            for q2 in range(4):
                row = slotb[pl.ds(q2 * 16, 16)]
                bq = jnp.sum(jnp.where(lane == 0, row, 0.0))
                iq = jnp.sum(jnp.where(lane == 1, plsc.bitcast(row, jnp.int32), 0))
                better = (bq > best) | ((bq == best) & (iq < bidx))
                best = jnp.where(better, bq, best)
                bidx = jnp.where(better, iq, bidx)
            active = best >= 0.0
            bidx = jnp.where(active, bidx, 0)
            idxv = jnp.zeros((16,), jnp.int32) + bidx
            wx1 = plsc.load_gather(cx1, [idxv])
            wy1 = plsc.load_gather(cy1, [idxv])
            wx2 = plsc.load_gather(cx2, [idxv])
            wy2 = plsc.load_gather(cy2, [idxv])
            war = plsc.load_gather(car, [idxv])
            bv = jnp.zeros((16,), jnp.float32) + best

            @pl.when((qq == 0) & active)
            def _():
                wcl = plsc.load_gather(ccls, [idxv])
                kv = jnp.zeros((16,), jnp.int32) + k
                l0 = lane == 0
                plsc.store_scatter(os_v, [kv], bv, mask=l0)
                plsc.store_scatter(oc_v, [kv], wcl, mask=l0)
                bvals = jnp.where(lane == 0, wx1,
                        jnp.where(lane == 1, wy1,
                        jnp.where(lane == 2, wx2, wy2)))
                plsc.store_scatter(ob_v, [kv * 4 + lane], bvals, mask=lane < 4)

            plsc.store_scatter(csc, [idxv], neg1, mask=lane == 0)

            def sup_body(j4, c2):
                b2, bj2 = c2
                for u in range(4):
                    j = j4 * 4 + u
                    s = csc[pl.ds(j * 16, 16)]
                    x1 = cx1[pl.ds(j * 16, 16)]
                    y1 = cy1[pl.ds(j * 16, 16)]
                    x2 = cx2[pl.ds(j * 16, 16)]
                    y2 = cy2[pl.ds(j * 16, 16)]
                    ar = car[pl.ds(j * 16, 16)]
                    xx1 = jnp.maximum(wx1, x1)
                    yy1 = jnp.maximum(wy1, y1)
                    xx2 = jnp.minimum(wx2, x2)
                    yy2 = jnp.minimum(wy2, y2)
                    w = jnp.maximum(xx2 - xx1, 0.0)
                    h = jnp.maximum(yy2 - yy1, 0.0)
                    inter = w * h
                    iou = inter / (war + ar - inter + 1e-12)
                    snew = jnp.where(iou > 0.6, neg1, s)
                    csc[pl.ds(j * 16, 16)] = snew
                    m = jnp.max(snew)
                    better = m > b2
                    b2 = jnp.where(better, m, b2)
                    bj2 = jnp.where(better, j, bj2)
                return b2, bj2

            b3, bj3 = lax.fori_loop(qq * (_QV // 4), (qq + 1) * (_QV // 4),
                                    sup_body,
                                    (jnp.float32(-2.0), jnp.int32(0)))
            publish(b3, bj3, 1 - p)
            plsc.subcore_barrier()
            return carry

        lax.fori_loop(0, 100, keep_body, 0)

        @pl.when(qq == 0)
        def _():
            pltpu.sync_copy(os_v, outs_hbm.at[img])
            pltpu.sync_copy(oc_v, outc_hbm.at[img])
            pltpu.sync_copy(ob_v, outb_hbm.at[img])

    return nms_kernel


_nms_call = _make_nms_kernel()


def kernel(cls_head_0, reg_head_0, center_head_0, positions_0,
           cls_head_1, reg_head_1, center_head_1, positions_1,
           cls_head_2, reg_head_2, center_head_2, positions_2,
           cls_head_3, reg_head_3, center_head_3, positions_3,
           cls_head_4, reg_head_4, center_head_4, positions_4):
    heads = [
        (cls_head_0, reg_head_0, center_head_0, positions_0),
        (cls_head_1, reg_head_1, center_head_1, positions_1),
        (cls_head_2, reg_head_2, center_head_2, positions_2),
        (cls_head_3, reg_head_3, center_head_3, positions_3),
        (cls_head_4, reg_head_4, center_head_4, positions_4),
    ]
    scs, cls_, xy1s, xy2s = [], [], [], []
    for l, (ch, rh, th, ph) in enumerate(heads):
        n = _LVL_N[l]
        b = ch.shape[0]
        sc, cl, xy1, xy2 = _decode_level(
            ch.reshape(b, n, ch.shape[-1]),
            rh.reshape(b, n, 4).transpose(0, 2, 1),
            th.reshape(b, n),
            ph.reshape(b, n, 2).transpose(0, 2, 1),
            n, _CHUNK[l])
        scs.append(sc)
        cls_.append(cl)
        xy1s.append(xy1)
        xy2s.append(xy2)
    thr = pl.pallas_call(
        _thresh_body,
        out_shape=jax.ShapeDtypeStruct((8, 128), jnp.int32),
    )(scs[0], scs[1], scs[2])
    pad_s = jnp.full((8, 64), -1.0, jnp.float32)
    pad_i = jnp.zeros((8, 64), jnp.int32)
    scs[4] = jnp.concatenate([scs[4], pad_s], axis=1)
    cls_[4] = jnp.concatenate([cls_[4], pad_s], axis=1)
    xy1s[4] = jnp.concatenate([xy1s[4], pad_i], axis=1)
    xy2s[4] = jnp.concatenate([xy2s[4], pad_i], axis=1)
    outs, outc, outb = _nms_call(*scs, *xy1s, *xy2s, *cls_, thr)
    outb = outb.reshape(8, _OUTP, 4)
    return outs[:, :100], outc[:, :100], outb[:, :100, :]


# speedup vs baseline: 2.8228x; 1.1189x over previous
import functools

import jax
import jax.numpy as jnp
from jax import lax
from jax.experimental import pallas as pl
from jax.experimental.pallas import tpu as pltpu
from jax.experimental.pallas import tpu_sc as plsc

_LVL_N = (16384, 4096, 1024, 256, 64)
_LVL_NP = (16384, 4096, 1024, 256, 128)
_CHUNK = (2048, 2048, 1024, 256, 64)
_NTOT = 21888
_NCAP = 3328
_NCV = _NCAP // 16
_QN = _NCAP // 4
_QV = _NCV // 4
_OUTP = 112


def _decode_body(cls_ref, reg_ref, ctr_ref, pos_ref,
                 sc_ref, cl_ref, xy1_ref, xy2_ref):
    c = cls_ref[...]
    m = jnp.max(c, axis=-1)
    cls_iota = lax.broadcasted_iota(jnp.int32, c.shape, 2)
    am = jnp.min(jnp.where(c == m[..., None], cls_iota, c.shape[-1]), axis=-1)
    t = ctr_ref[...]
    score = jnp.sqrt(jax.nn.sigmoid(m) * jax.nn.sigmoid(t))
    r = jnp.exp(reg_ref[...])
    p = pos_ref[...]
    xmin = p[:, 0, :] - r[:, 0, :]
    ymin = p[:, 1, :] - r[:, 1, :]
    xmax = p[:, 0, :] + r[:, 2, :]
    ymax = p[:, 1, :] + r[:, 3, :]
    f = lambda v: v.astype(jnp.int32).astype(jnp.float32)
    x1 = jnp.maximum(f(xmin), 0.0)
    y1 = jnp.maximum(f(ymin), 0.0)
    x2 = jnp.minimum(f(xmax), 1023.0)
    y2 = jnp.minimum(f(ymax), 1023.0)
    sc_ref[...] = score
    cl_ref[...] = am.astype(jnp.float32)
    pk = lambda a, b: (a.astype(jnp.int32) & 0xFFFF) | (b.astype(jnp.int32) << 16)
    xy1_ref[...] = pk(x1, y1)
    xy2_ref[...] = pk(x2, y2)


def _decode_level(ch, rh, th, ph, n, chunk):
    b = ch.shape[0]
    grid = (n // chunk,)
    return pl.pallas_call(
        _decode_body,
        grid=grid,
        in_specs=[
            pl.BlockSpec((b, chunk, 80), lambda j: (0, j, 0)),
            pl.BlockSpec((b, 4, chunk), lambda j: (0, 0, j)),
            pl.BlockSpec((b, chunk), lambda j: (0, j)),
            pl.BlockSpec((b, 2, chunk), lambda j: (0, 0, j)),
        ],
        out_specs=[
            pl.BlockSpec((b, chunk), lambda j: (0, j)),
            pl.BlockSpec((b, chunk), lambda j: (0, j)),
            pl.BlockSpec((b, chunk), lambda j: (0, j)),
            pl.BlockSpec((b, chunk), lambda j: (0, j)),
        ],
        out_shape=[
            jax.ShapeDtypeStruct((b, n), jnp.float32),
            jax.ShapeDtypeStruct((b, n), jnp.float32),
            jax.ShapeDtypeStruct((b, n), jnp.int32),
            jax.ShapeDtypeStruct((b, n), jnp.int32),
        ],
    )(ch, rh, th, ph)


def _thresh_body(s0_ref, s1_ref, s2_ref, out_ref):
    lanes = lax.broadcasted_iota(jnp.int32, (8, 128), 1)
    refs = (s0_ref, s1_ref, s2_ref)

    def it_body(_, carry):
        lo, hi = carry
        mid = lax.shift_right_arithmetic(lo + hi + 1, 1)
        tf = lax.bitcast_convert_type(mid, jnp.float32)
        for l in range(3):
            t = lax.slice(tf, (0, l), (8, l + 1))
            s = refs[l][...]
            cnt = jnp.sum((s >= t).astype(jnp.int32), axis=1, keepdims=True)
            ge = cnt >= 1000
            lm = lanes == l
            lo = jnp.where(lm & ge, mid, lo)
            hi = jnp.where(lm & (~ge), mid - 1, hi)
        return lo, hi

    lo0 = jnp.zeros((8, 128), jnp.int32)
    hi0 = jnp.where(lanes < 3, 0x3F800000, 0)
    lo, _ = lax.fori_loop(0, 31, it_body, (lo0, hi0))
    acc = lo
    tfv = lax.bitcast_convert_type(lo, jnp.float32)
    for l in range(3):
        t = lax.slice(tfv, (0, l), (8, l + 1))
        s = refs[l][...]
        cgt = jnp.sum((s > t).astype(jnp.int32), axis=1, keepdims=True)
        acc = acc + jnp.where(lanes == (3 + l), cgt, 0)
    out_ref[...] = acc


def _make_nms_kernel():
    mesh = plsc.VectorSubcoreMesh(core_axis_name="c", subcore_axis_name="s")

    @functools.partial(
        pl.kernel,
        mesh=mesh,
        compiler_params=pltpu.CompilerParams(needs_layout_passes=False),
        out_type=[
            jax.ShapeDtypeStruct((8, _OUTP), jnp.float32),
            jax.ShapeDtypeStruct((8, _OUTP), jnp.float32),
            jax.ShapeDtypeStruct((8, _OUTP * 4), jnp.float32),
        ],
        scratch_types=[
            pltpu.VMEM((_NTOT,), jnp.float32),
            pltpu.VMEM((_NTOT,), jnp.int32),
            pltpu.VMEM((_NTOT,), jnp.int32),
            pltpu.VMEM((_NTOT,), jnp.float32),
            pltpu.VMEM((128,), jnp.int32),
            pltpu.VMEM((_NCAP,), jnp.float32),
            pltpu.VMEM((_NCAP,), jnp.int32),
            pltpu.VMEM((_NCAP,), jnp.int32),
            pltpu.VMEM((_NCAP,), jnp.float32),
            pltpu.VMEM((_NCAP,), jnp.float32),
            pltpu.VMEM((_NCAP,), jnp.float32),
            pltpu.VMEM((_NCAP,), jnp.float32),
            pltpu.VMEM((_NCAP,), jnp.float32),
            pltpu.VMEM((_NCAP,), jnp.float32),
            pltpu.VMEM((_OUTP,), jnp.float32),
            pltpu.VMEM((_OUTP,), jnp.float32),
            pltpu.VMEM((_OUTP * 4,), jnp.float32),
        ],
    )
    def nms_kernel(sc0, sc1, sc2, sc3, sc4,
                   xa0, xa1, xa2, xa3, xa4,
                   xb0, xb1, xb2, xb3, xb4,
                   cl0, cl1, cl2, cl3, cl4,
                   thr_hbm,
                   outs_hbm, outc_hbm, outb_hbm,
                   sc_v, xy1_v, xy2_v, cls_v, thr_v,
                   csc, cxy1, cxy2, ccls,
                   cx1, cy1, cx2, cy2, car,
                   os_v, oc_v, ob_v):
        wid = lax.axis_index("s") * 2 + lax.axis_index("c")

        lane = lax.iota(jnp.int32, 16)
        neg1 = jnp.full((16,), -1.0, jnp.float32)

        def init_csc(j4, carry):
            for u in range(4):
                csc[pl.ds((j4 * 4 + u) * 16, 16)] = neg1
            return carry

        @pl.when(wid < 8)
        def _():
            i = wid
            off = 0
            for l, (shb, xahb, xbhb, clhb) in enumerate(
                    zip((sc0, sc1, sc2, sc3, sc4),
                        (xa0, xa1, xa2, xa3, xa4),
                        (xb0, xb1, xb2, xb3, xb4),
                        (cl0, cl1, cl2, cl3, cl4))):
                nl = _LVL_NP[l]
                pltpu.sync_copy(shb.at[i], sc_v.at[pl.ds(off, nl)])
                pltpu.sync_copy(xahb.at[i], xy1_v.at[pl.ds(off, nl)])
                pltpu.sync_copy(xbhb.at[i], xy2_v.at[pl.ds(off, nl)])
                pltpu.sync_copy(clhb.at[i], cls_v.at[pl.ds(off, nl)])
                off += nl
            pltpu.sync_copy(thr_hbm.at[i], thr_v)

            lax.fori_loop(0, _NCV // 4, init_csc, 0)

            def init_out(j, carry):
                os_v[pl.ds(j * 16, 16)] = neg1
                oc_v[pl.ds(j * 16, 16)] = neg1
                return carry
            lax.fori_loop(0, _OUTP // 16, init_out, 0)

            def init_ob(j, carry):
                ob_v[pl.ds(j * 16, 16)] = neg1
                return carry
            lax.fori_loop(0, _OUTP * 4 // 16, init_ob, 0)

            tv = thr_v[pl.ds(0, 16)]

            def lane_scalar(vec, l):
                return jnp.sum(jnp.where(lane == l, vec, 0))

            voff = 0
            woff = jnp.int32(0)
            for l, nl in enumerate(_LVL_NP):
                nv = nl // 16
                if nl >= 1000:
                    tbits = lane_scalar(tv, l)
                    ngt = lane_scalar(tv, 3 + l)
                    budget0 = jnp.int32(1000) - ngt
                    tf = lax.bitcast_convert_type(
                        jnp.zeros((16,), jnp.int32) + tbits, jnp.float32)

                    def scan_body(j2, carry, tf=tf):
                        w, budget = carry
                        for u in range(2):
                            j = j2 * 2 + u
                            s = sc_v[pl.ds(j * 16, 16)]
                            gt = s > tf
                            eq = s == tf
                            eqc = jnp.cumsum(eq.astype(jnp.int32))
                            sel_eq = eq & (eqc <= (jnp.zeros((16,), jnp.int32) + budget))
                            sel = gt | sel_eq
                            budget = budget - jnp.sum(sel_eq.astype(jnp.int32))
                            keep = sel & (s > 0.01)
                            ki = keep.astype(jnp.int32)
                            idx = (jnp.zeros((16,), jnp.int32) + w) + jnp.cumsum(ki) - 1
                            plsc.store_scatter(csc, [idx], s, mask=keep)
                            plsc.store_scatter(cxy1, [idx], xy1_v[pl.ds(j * 16, 16)], mask=keep)
                            plsc.store_scatter(cxy2, [idx], xy2_v[pl.ds(j * 16, 16)], mask=keep)
                            plsc.store_scatter(ccls, [idx], cls_v[pl.ds(j * 16, 16)], mask=keep)
                            w = w + jnp.sum(ki)
                        return w, budget

                    woff, _ = lax.fori_loop(voff // 2, (voff + nv) // 2,
                                            scan_body, (woff, budget0))
                else:
                    def scan_small(j2, w):
                        for u in range(2):
                            j = j2 * 2 + u
                            s = sc_v[pl.ds(j * 16, 16)]
                            keep = s > 0.01
                            ki = keep.astype(jnp.int32)
                            idx = (jnp.zeros((16,), jnp.int32) + w) + jnp.cumsum(ki) - 1
                            plsc.store_scatter(csc, [idx], s, mask=keep)
                            plsc.store_scatter(cxy1, [idx], xy1_v[pl.ds(j * 16, 16)], mask=keep)
                            plsc.store_scatter(cxy2, [idx], xy2_v[pl.ds(j * 16, 16)], mask=keep)
                            plsc.store_scatter(ccls, [idx], cls_v[pl.ds(j * 16, 16)], mask=keep)
                            w = w + jnp.sum(ki)
                        return w

                    woff = lax.fori_loop(voff // 2, (voff + nv) // 2,
                                         scan_small, woff)
                voff += nv

            def unpack_body(j4, carry):
                for u in range(4):
                    j = j4 * 4 + u
                    p1 = cxy1[pl.ds(j * 16, 16)]
                    p2 = cxy2[pl.ds(j * 16, 16)]
                    x1 = lax.shift_right_arithmetic(lax.shift_left(p1, 16), 16).astype(jnp.float32)
                    y1 = lax.shift_right_arithmetic(p1, 16).astype(jnp.float32)
                    x2 = lax.shift_right_arithmetic(lax.shift_left(p2, 16), 16).astype(jnp.float32)
                    y2 = lax.shift_right_arithmetic(p2, 16).astype(jnp.float32)
                    cx1[pl.ds(j * 16, 16)] = x1
                    cy1[pl.ds(j * 16, 16)] = y1
                    cx2[pl.ds(j * 16, 16)] = x2
                    cy2[pl.ds(j * 16, 16)] = y2
                    car[pl.ds(j * 16, 16)] = (x2 - x1) * (y2 - y1)
                return carry
            lax.fori_loop(0, _NCV // 4, unpack_body, 0)

            def amax_body(j4, carry):
                b, bj = carry
                for u in range(4):
                    j = j4 * 4 + u
                    m = jnp.max(csc[pl.ds(j * 16, 16)])
                    better = m > b
                    b = jnp.where(better, m, b)
                    bj = jnp.where(better, j, bj)
                return b, bj

            best0, bestj0 = lax.fori_loop(0, _NCV // 4, amax_body,
                                          (jnp.float32(-2.0), jnp.int32(0)))

            def cond_fn(st):
                kept, b, bj = st
                return (kept < 100) & (b >= 0.0)

            def body_fn(st):
                kept, b, bj = st
                bv = jnp.zeros((16,), jnp.float32) + b
                v = csc[pl.ds(bj * 16, 16)]
                eqm = v == bv
                lidx = jnp.zeros((16,), jnp.int32) + plsc.all_reduce_ffs(eqm)
                idxv = (jnp.zeros((16,), jnp.int32) + bj * 16) + lidx
                wx1 = plsc.load_gather(cx1, [idxv])
                wy1 = plsc.load_gather(cy1, [idxv])
                wx2 = plsc.load_gather(cx2, [idxv])
                wy2 = plsc.load_gather(cy2, [idxv])
                war = plsc.load_gather(car, [idxv])
                wcl = plsc.load_gather(ccls, [idxv])
                l0 = lane == 0
                kv = jnp.zeros((16,), jnp.int32) + kept
                plsc.store_scatter(os_v, [kv], bv, mask=l0)
                plsc.store_scatter(oc_v, [kv], wcl, mask=l0)
                bvals = jnp.where(lane == 0, wx1,
                        jnp.where(lane == 1, wy1,
                        jnp.where(lane == 2, wx2, wy2)))
                plsc.store_scatter(ob_v, [kv * 4 + lane], bvals, mask=lane < 4)
                plsc.store_scatter(csc, [idxv], neg1, mask=l0)

                def sup_body(j4, c2):
                    b2, bj2 = c2
                    for u in range(4):
                        j = j4 * 4 + u
                        s = csc[pl.ds(j * 16, 16)]
                        x1 = cx1[pl.ds(j * 16, 16)]
                        y1 = cy1[pl.ds(j * 16, 16)]
                        x2 = cx2[pl.ds(j * 16, 16)]
                        y2 = cy2[pl.ds(j * 16, 16)]
                        ar = car[pl.ds(j * 16, 16)]
                        xx1 = jnp.maximum(wx1, x1)
                        yy1 = jnp.maximum(wy1, y1)
                        xx2 = jnp.minimum(wx2, x2)
                        yy2 = jnp.minimum(wy2, y2)
                        w = jnp.maximum(xx2 - xx1, 0.0)
                        h = jnp.maximum(yy2 - yy1, 0.0)
                        inter = w * h
                        iou = inter / (war + ar - inter + 1e-12)
                        snew = jnp.where(iou > 0.6, neg1, s)
                        csc[pl.ds(j * 16, 16)] = snew
                        m = jnp.max(snew)
                        better = m > b2
                        b2 = jnp.where(better, m, b2)
                        bj2 = jnp.where(better, j, bj2)
                    return b2, bj2

                b3, bj3 = lax.fori_loop(0, _NCV // 4, sup_body,
                                        (jnp.float32(-2.0), jnp.int32(0)))
                return kept + 1, b3, bj3

            lax.while_loop(cond_fn, body_fn, (jnp.int32(0), best0, bestj0))

            pltpu.sync_copy(os_v, outs_hbm.at[i])
            pltpu.sync_copy(oc_v, outc_hbm.at[i])
            pltpu.sync_copy(ob_v, outb_hbm.at[i])

    return nms_kernel


_nms_call = _make_nms_kernel()


def kernel(cls_head_0, reg_head_0, center_head_0, positions_0,
           cls_head_1, reg_head_1, center_head_1, positions_1,
           cls_head_2, reg_head_2, center_head_2, positions_2,
           cls_head_3, reg_head_3, center_head_3, positions_3,
           cls_head_4, reg_head_4, center_head_4, positions_4):
    heads = [
        (cls_head_0, reg_head_0, center_head_0, positions_0),
        (cls_head_1, reg_head_1, center_head_1, positions_1),
        (cls_head_2, reg_head_2, center_head_2, positions_2),
        (cls_head_3, reg_head_3, center_head_3, positions_3),
        (cls_head_4, reg_head_4, center_head_4, positions_4),
    ]
    scs, cls_, xy1s, xy2s = [], [], [], []
    for l, (ch, rh, th, ph) in enumerate(heads):
        n = _LVL_N[l]
        b = ch.shape[0]
        sc, cl, xy1, xy2 = _decode_level(
            ch.reshape(b, n, ch.shape[-1]),
            rh.reshape(b, n, 4).transpose(0, 2, 1),
            th.reshape(b, n),
            ph.reshape(b, n, 2).transpose(0, 2, 1),
            n, _CHUNK[l])
        scs.append(sc)
        cls_.append(cl)
        xy1s.append(xy1)
        xy2s.append(xy2)
    thr = pl.pallas_call(
        _thresh_body,
        out_shape=jax.ShapeDtypeStruct((8, 128), jnp.int32),
    )(scs[0], scs[1], scs[2])
    pad_s = jnp.full((8, 64), -1.0, jnp.float32)
    pad_i = jnp.zeros((8, 64), jnp.int32)
    scs[4] = jnp.concatenate([scs[4], pad_s], axis=1)
    cls_[4] = jnp.concatenate([cls_[4], pad_s], axis=1)
    xy1s[4] = jnp.concatenate([xy1s[4], pad_i], axis=1)
    xy2s[4] = jnp.concatenate([xy2s[4], pad_i], axis=1)
    outs, outc, outb = _nms_call(*scs, *xy1s, *xy2s, *cls_, thr)
    outb = outb.reshape(8, _OUTP, 4)
    return outs[:, :100], outc[:, :100], outb[:, :100, :]
